# Initial kernel scaffold; baseline (speedup 1.0000x reference)
import jax
import jax.numpy as jnp
from jax.experimental import pallas as pl


def kernel(inputs, q_table, r_table):
    raise NotImplementedError("write your pallas kernel here")


# speedup vs baseline: 3.2157x; 3.2157x over previous
import functools

import jax
import jax.numpy as jnp
from jax import lax
from jax.experimental import pallas as pl
from jax.experimental.pallas import tpu as pltpu
from jax.experimental.pallas import tpu_sc as plsc

_BUCKETS = 320
_EMBED = 64
_NC = 2
_NS = 16
_NW = _NC * _NS
_LANES = 16


def _qr_body(total, per_w, chunk, idx_hbm, qt_hbm, rt_hbm, out_hbm,
             qt_v, rt_v, idx_v, buf0, buf1, sem0, sem1):
  nchunk = per_w // chunk
  groups = chunk // _LANES
  wid = lax.axis_index("s") * _NC + lax.axis_index("c")
  base = wid * per_w

  pltpu.sync_copy(qt_hbm, qt_v)
  pltpu.sync_copy(rt_hbm, rt_v)
  pltpu.sync_copy(idx_hbm.at[pl.ds(base, per_w)], idx_v)

  lane64 = lax.iota(jnp.int32, _LANES) * _EMBED

  def compute(c, buf):
    @plsc.parallel_loop(0, groups)
    def group(g):
      iv = idx_v[pl.ds(c * chunk + g * _LANES, _LANES)]
      x = lax.shift_right_logical(iv, 6)
      q = lax.shift_right_logical(x * 6554, 15)
      r = iv - q * _BUCKETS
      qb = q * _EMBED
      rb = r * _EMBED
      ob = lane64 + g * (_LANES * _EMBED)
      for d in range(_EMBED):
        qv = plsc.load_gather(qt_v, [qb + d])
        rv = plsc.load_gather(rt_v, [rb + d])
        plsc.store_scatter(buf, [ob + d], qv * rv)

  def start_copy(c, buf, sem):
    cp = pltpu.make_async_copy(
        buf, out_hbm.at[pl.ds((base + c * chunk) * _EMBED, chunk * _EMBED)],
        sem)
    cp.start()

  def drain(buf, sem):
    pltpu.make_async_copy(
        buf, out_hbm.at[pl.ds(base * _EMBED, chunk * _EMBED)], sem).wait()

  compute(0, buf0)
  start_copy(0, buf0, sem0)
  compute(1, buf1)
  start_copy(1, buf1, sem1)

  def pair(p, carry):
    c0 = 2 * p
    drain(buf0, sem0)
    compute(c0, buf0)
    start_copy(c0, buf0, sem0)
    drain(buf1, sem1)
    compute(c0 + 1, buf1)
    start_copy(c0 + 1, buf1, sem1)
    return carry

  lax.fori_loop(1, nchunk // 2, pair, 0)
  drain(buf0, sem0)
  drain(buf1, sem1)


def kernel(inputs, q_table, r_table):
  total = inputs.shape[0] * inputs.shape[1]
  per_w = total // _NW
  chunk = 400
  idx = jnp.reshape(inputs, (total,)).astype(jnp.int32)
  qt = jnp.reshape(q_table, (_BUCKETS * _EMBED,))
  rt = jnp.reshape(r_table, (_BUCKETS * _EMBED,))

  mesh = plsc.VectorSubcoreMesh(core_axis_name="c", subcore_axis_name="s")
  body = functools.partial(_qr_body, total, per_w, chunk)
  out = pl.kernel(
      body,
      out_type=jax.ShapeDtypeStruct((total * _EMBED,), jnp.float32),
      mesh=mesh,
      compiler_params=pltpu.CompilerParams(needs_layout_passes=False),
      scratch_types=[
          pltpu.VMEM((_BUCKETS * _EMBED,), jnp.float32),
          pltpu.VMEM((_BUCKETS * _EMBED,), jnp.float32),
          pltpu.VMEM((per_w,), jnp.int32),
          pltpu.VMEM((chunk * _EMBED,), jnp.float32),
          pltpu.VMEM((chunk * _EMBED,), jnp.float32),
          pltpu.SemaphoreType.DMA,
          pltpu.SemaphoreType.DMA,
      ],
  )(idx, qt, rt)
  return jnp.reshape(out, (inputs.shape[0], inputs.shape[1], _EMBED))


# speedup vs baseline: 8.6099x; 2.6775x over previous
import functools

import jax
import jax.numpy as jnp
from jax import lax
from jax.experimental import pallas as pl
from jax.experimental.pallas import tpu as pltpu
from jax.experimental.pallas import tpu_sc as plsc

_BUCKETS = 320
_EMBED = 64
_NC = 2
_NS = 16
_NW = _NC * _NS
_LANES = 16


def _qr_body(total, per_w, chunk, idx_hbm, qt_hbm, rt_hbm, out_hbm,
             qt_v, rt_v, idx_v, buf0, buf1, sem0, sem1):
  nchunk = per_w // chunk
  wid = lax.axis_index("s") * _NC + lax.axis_index("c")
  base = wid * per_w

  pltpu.sync_copy(qt_hbm, qt_v)
  pltpu.sync_copy(rt_hbm, rt_v)
  pltpu.sync_copy(idx_hbm.at[pl.ds(base, per_w)], idx_v)

  def compute(c, buf):
    @plsc.parallel_loop(0, chunk // _LANES, unroll=2)
    def group(g):
      iv = idx_v[pl.ds(c * chunk + g * _LANES, _LANES)]
      q = lax.shift_right_logical(lax.shift_right_logical(iv, 6) * 6554, 15)
      r = iv - q * _BUCKETS
      qb = q * _EMBED
      rb = r * _EMBED
      for l in range(_LANES):
        qbl = qb[l]
        rbl = rb[l]
        ob = (g * _LANES + l) * _EMBED
        for t in range(_EMBED // _LANES):
          qv = qt_v[pl.ds(qbl + t * _LANES, _LANES)]
          rv = rt_v[pl.ds(rbl + t * _LANES, _LANES)]
          buf[pl.ds(ob + t * _LANES, _LANES)] = qv * rv

  def start_copy(c, buf, sem):
    cp = pltpu.make_async_copy(
        buf, out_hbm.at[pl.ds((base + c * chunk) * _EMBED, chunk * _EMBED)],
        sem)
    cp.start()

  def drain(buf, sem):
    pltpu.make_async_copy(
        buf, out_hbm.at[pl.ds(base * _EMBED, chunk * _EMBED)], sem).wait()

  compute(0, buf0)
  start_copy(0, buf0, sem0)
  compute(1, buf1)
  start_copy(1, buf1, sem1)

  def pair(p, carry):
    c0 = 2 * p
    drain(buf0, sem0)
    compute(c0, buf0)
    start_copy(c0, buf0, sem0)
    drain(buf1, sem1)
    compute(c0 + 1, buf1)
    start_copy(c0 + 1, buf1, sem1)
    return carry

  lax.fori_loop(1, nchunk // 2, pair, 0)
  drain(buf0, sem0)
  drain(buf1, sem1)


def kernel(inputs, q_table, r_table):
  total = inputs.shape[0] * inputs.shape[1]
  per_w = total // _NW
  chunk = 400
  idx = jnp.reshape(inputs, (total,)).astype(jnp.int32)
  qt = jnp.reshape(q_table, (_BUCKETS * _EMBED,))
  rt = jnp.reshape(r_table, (_BUCKETS * _EMBED,))

  mesh = plsc.VectorSubcoreMesh(core_axis_name="c", subcore_axis_name="s")
  body = functools.partial(_qr_body, total, per_w, chunk)
  out = pl.kernel(
      body,
      out_type=jax.ShapeDtypeStruct((total * _EMBED,), jnp.float32),
      mesh=mesh,
      compiler_params=pltpu.CompilerParams(needs_layout_passes=False),
      scratch_types=[
          pltpu.VMEM((_BUCKETS * _EMBED,), jnp.float32),
          pltpu.VMEM((_BUCKETS * _EMBED,), jnp.float32),
          pltpu.VMEM((per_w,), jnp.int32),
          pltpu.VMEM((chunk * _EMBED,), jnp.float32),
          pltpu.VMEM((chunk * _EMBED,), jnp.float32),
          pltpu.SemaphoreType.DMA,
          pltpu.SemaphoreType.DMA,
      ],
  )(idx, qt, rt)
  return jnp.reshape(out, (inputs.shape[0], inputs.shape[1], _EMBED))
